# baseline (speedup 1.0000x reference)
import jax
import jax.numpy as jnp
from jax import lax
from jax.experimental import pallas as pl
from jax.experimental.pallas import tpu as pltpu
from jax.experimental.pallas import tpu_sc as plsc

B, T = 16384, 200
V, D = 10, 8
N = B * T
NW = 32
JB = B // 128 // NW
NCHUNK = T // 8
XC = JB * 1024
OC = XC * D
NBUF = 2


def _sc_embed(x_hbm, emb_hbm, out_hbm, x_v0, x_v1, o_v0, o_v1, emb_v,
              si0, si1, so0, so1):
    wid = lax.axis_index("s") * 2 + lax.axis_index("c")
    pltpu.sync_copy(emb_hbm, emb_v)
    tab = [emb_v[pl.ds(d * 16, 16)] for d in range(D)]
    xb = (x_v0, x_v1)
    ob = (o_v0, o_v1)
    si = (si0, si1)
    so = (so0, so1)

    def in_copy(ci, b):
        return pltpu.make_async_copy(
            x_hbm.at[pl.ds(ci * (128 * 1024) + wid * XC, XC)], xb[b], si[b])

    def out_copies(ci, b):
        return [
            pltpu.make_async_copy(
                ob[b].at[pl.ds(tr * XC, XC)],
                out_hbm.at[pl.ds(ci * (8 * 128 * 1024) + tr * (128 * 1024)
                                 + wid * XC, XC)],
                so[b])
            for tr in range(8)
        ]

    def process(ci, b):
        in_copy(ci, b).wait()

        @pl.when(ci >= NBUF)
        def _wait_out():
            for cp in out_copies(ci - NBUF, b):
                cp.wait()

        x_v = xb[b]
        o_v = ob[b]

        @plsc.parallel_loop(0, XC // 16, unroll=8)
        def _grp(q):
            xv = x_v[pl.ds(q * 16, 16)]
            off = ((q >> 3) & 7) * XC + (q >> 6) * 1024 + (q & 7) * 16
            for d in range(D):
                vals = jnp.take_along_axis(tab[d], xv, axis=0)
                o_v[pl.ds(off + d * 128, 16)] = vals

        for cp in out_copies(ci, b):
            cp.start()

        @pl.when(ci + NBUF < NCHUNK)
        def _next_in():
            in_copy(ci + NBUF, b).start()

    in_copy(0, 0).start()
    in_copy(1, 1).start()

    def pair(ci2, _):
        for b in range(NBUF):
            process(ci2 * NBUF + b, b)
        return 0

    lax.fori_loop(0, (NCHUNK - 1) // NBUF, pair, 0)
    process(NCHUNK - 1, 0)
    for cp in out_copies(NCHUNK - 2, 1):
        cp.wait()
    for cp in out_copies(NCHUNK - 1, 0):
        cp.wait()


def kernel(x, emb):
    xr = (x.astype(jnp.int32)
          .reshape(128, 128, 25, 8)
          .transpose(2, 0, 3, 1)
          .reshape(-1))
    ef = jnp.pad(emb.T, ((0, 0), (0, 6))).reshape(-1)
    mesh = plsc.VectorSubcoreMesh(core_axis_name="c", subcore_axis_name="s")
    run = pl.kernel(
        _sc_embed,
        out_type=jax.ShapeDtypeStruct((N * D,), jnp.float32),
        mesh=mesh,
        compiler_params=pltpu.CompilerParams(needs_layout_passes=False),
        scratch_types=[
            pltpu.VMEM((XC,), jnp.int32),
            pltpu.VMEM((XC,), jnp.int32),
            pltpu.VMEM((OC,), jnp.float32),
            pltpu.VMEM((OC,), jnp.float32),
            pltpu.VMEM((D * 16,), jnp.float32),
            pltpu.SemaphoreType.DMA,
            pltpu.SemaphoreType.DMA,
            pltpu.SemaphoreType.DMA,
            pltpu.SemaphoreType.DMA,
        ],
    )
    out = run(xr, ef)
    return (
        out.reshape(T, B // 128, D, 128)
        .transpose(1, 3, 0, 2)
        .reshape(B, T, D)
    )
